# Initial kernel scaffold; baseline (speedup 1.0000x reference)
import jax
import jax.numpy as jnp
from jax.experimental import pallas as pl


def kernel(node_obs, adj, W1, b1, W2, b2, Wq, bq, Wk, bk, Wv, bv, We, Wskip, bskip):
    raise NotImplementedError("write your pallas kernel here")


# speedup vs baseline: 102.1757x; 102.1757x over previous
import functools

import numpy as np
import jax
import jax.numpy as jnp
from jax.experimental import pallas as pl

_RADIUS = 0.5
_H = 4
_C = 32
_G = 8


def _gnn_kernel(x_ref, a_ref, w1d_ref, w1e_ref, b1_ref, w2_ref, b2_ref,
                wcat_ref, bcat_ref, we_ref, hs_ref, hst_ref, out_ref):
    G, N, D = x_ref.shape
    EH = w2_ref.shape[0]
    HC = we_ref.shape[1]
    f32 = jnp.float32
    hi = jax.lax.Precision.HIGHEST

    a = a_ref[...]
    mask = (a > 0.0) & (a < _RADIUS)
    maskf = mask.astype(f32)
    am = a * maskf

    x2 = x_ref[...].reshape(G * N, D)
    t1 = jnp.dot(x2, w1d_ref[...], precision=hi) + b1_ref[...]
    w1e = w1e_ref[...].reshape(1, 1, 1, EH)
    h1 = jnp.maximum(t1.reshape(G, N, 1, EH) + am[..., None] * w1e, 0.0)
    h2 = jnp.maximum(
        jnp.dot(h1.reshape(G * N * N, EH), w2_ref[...], precision=hi)
        + b2_ref[...], 0.0).reshape(G, N, N, EH)
    h2 = h2 * maskf[..., None]
    x_emb = jnp.sum(h2, axis=1).reshape(G * N, EH)

    proj = jnp.dot(x_emb, wcat_ref[...], precision=hi) + bcat_ref[...]
    q = proj[:, 0 * HC:1 * HC].reshape(G, N, HC)
    k = proj[:, 1 * HC:2 * HC].reshape(G, N, HC)
    v = proj[:, 2 * HC:3 * HC].reshape(G, N, HC)
    skip = proj[:, 3 * HC:4 * HC]

    we = we_ref[...]
    hs = hs_ref[...]
    hst = hst_ref[...]

    qwe = jnp.dot(q.reshape(G * N, HC) * we, hs,
                  precision=hi).reshape(G, 1, N, _H)
    prod = k[:, :, None, :] * q[:, None, :, :]
    logits4 = jnp.dot(prod.reshape(G * N * N, HC), hs,
                      precision=hi).reshape(G, N, N, _H)
    scale = 1.0 / np.sqrt(float(_C))
    logits = (logits4 + am[..., None] * qwe) * scale

    maskf4 = maskf[..., None]
    lm = logits + (maskf4 - 1.0) * 1e30
    amax = jnp.max(lm, axis=1, keepdims=True)
    amax = jnp.where(amax > -1e29, amax, 0.0)
    ex = jnp.exp(logits - amax) * maskf4
    den = jnp.sum(ex, axis=1, keepdims=True)
    den = jnp.where(den > 0.0, den, 1.0)
    alpha = ex / den

    alpha128 = jnp.dot(alpha.reshape(G * N * N, _H), hst,
                       precision=hi).reshape(G, N, N, HC)
    msg = alpha128 * (v[:, :, None, :] + am[..., None] * we.reshape(1, 1, 1, HC))
    out1 = jnp.sum(msg, axis=1).reshape(G * N, HC)

    out_ref[...] = jnp.maximum(out1 + skip, 0.0)


@functools.partial(jax.jit, static_argnames=())
def kernel(node_obs, adj, W1, b1, W2, b2, Wq, bq, Wk, bk, Wv, bv, We,
           Wskip, bskip):
    M, N, D = node_obs.shape
    EH = W2.shape[0]
    HC = We.shape[1]
    G = _G

    w1d = W1[:D]
    w1e = W1[D:D + 1]
    wcat = jnp.concatenate([Wq, Wk, Wv, Wskip], axis=1)
    bcat = jnp.concatenate([bq, bk, bv, bskip]).reshape(1, 4 * HC)
    hs = jnp.asarray(np.kron(np.eye(_H), np.ones((_C, 1))), dtype=jnp.float32)
    hst = jnp.asarray(np.kron(np.eye(_H), np.ones((1, _C))), dtype=jnp.float32)

    full = lambda shape: pl.BlockSpec(shape, lambda i: (0,) * len(shape))
    out = pl.pallas_call(
        _gnn_kernel,
        grid=(M // G,),
        in_specs=[
            pl.BlockSpec((G, N, D), lambda i: (i, 0, 0)),
            pl.BlockSpec((G, N, N), lambda i: (i, 0, 0)),
            full((D, EH)),
            full((1, EH)),
            full((1, EH)),
            full((EH, EH)),
            full((1, EH)),
            full((EH, 4 * HC)),
            full((1, 4 * HC)),
            full((1, HC)),
            full((HC, _H)),
            full((_H, HC)),
        ],
        out_specs=pl.BlockSpec((G * N, HC), lambda i: (i, 0)),
        out_shape=jax.ShapeDtypeStruct((M * N, HC), jnp.float32),
    )(node_obs, adj, w1d, w1e, b1.reshape(1, EH), W2, b2.reshape(1, EH),
      wcat, bcat, We, hs, hst)
    return out


# speedup vs baseline: 348.9489x; 3.4152x over previous
import functools

import numpy as np
import jax
import jax.numpy as jnp
from jax.experimental import pallas as pl

_RADIUS = 0.5
_H = 4
_C = 32
_G = 8


def _gnn_kernel(x_ref, a_ref, w1d_ref, w1e_ref, b1_ref, w2_ref, b2_ref,
                wcat_ref, bcat_ref, we_ref, hs_ref, hst_ref, out_ref):
    G, N, D = x_ref.shape
    EH = w2_ref.shape[0]
    HC = we_ref.shape[1]
    f32 = jnp.float32
    hi = jax.lax.Precision.DEFAULT

    a = a_ref[...]
    mask = (a > 0.0) & (a < _RADIUS)
    maskf = mask.astype(f32)
    am = a * maskf

    x2 = x_ref[...].reshape(G * N, D)
    t1 = jnp.dot(x2, w1d_ref[...], precision=hi) + b1_ref[...]
    w1e = w1e_ref[...].reshape(1, 1, 1, EH)
    h1 = jnp.maximum(t1.reshape(G, N, 1, EH) + am[..., None] * w1e, 0.0)
    h2 = jnp.maximum(
        jnp.dot(h1.reshape(G * N * N, EH), w2_ref[...], precision=hi)
        + b2_ref[...], 0.0).reshape(G, N, N, EH)
    h2 = h2 * maskf[..., None]
    x_emb = jnp.sum(h2, axis=1).reshape(G * N, EH)

    proj = jnp.dot(x_emb, wcat_ref[...], precision=hi) + bcat_ref[...]
    q = proj[:, 0 * HC:1 * HC].reshape(G, N, HC)
    k = proj[:, 1 * HC:2 * HC].reshape(G, N, HC)
    v = proj[:, 2 * HC:3 * HC].reshape(G, N, HC)
    skip = proj[:, 3 * HC:4 * HC]

    we = we_ref[...]
    hs = hs_ref[...]
    hst = hst_ref[...]

    qwe = jnp.dot(q.reshape(G * N, HC) * we, hs,
                  precision=hi).reshape(G, 1, N, _H)
    prod = k[:, :, None, :] * q[:, None, :, :]
    logits4 = jnp.dot(prod.reshape(G * N * N, HC), hs,
                      precision=hi).reshape(G, N, N, _H)
    scale = 1.0 / np.sqrt(float(_C))
    logits = (logits4 + am[..., None] * qwe) * scale

    maskf4 = maskf[..., None]
    lm = logits + (maskf4 - 1.0) * 1e30
    amax = jnp.max(lm, axis=1, keepdims=True)
    amax = jnp.where(amax > -1e29, amax, 0.0)
    ex = jnp.exp(logits - amax) * maskf4
    den = jnp.sum(ex, axis=1, keepdims=True)
    den = jnp.where(den > 0.0, den, 1.0)
    alpha = ex / den

    alpha128 = jnp.dot(alpha.reshape(G * N * N, _H), hst,
                       precision=hi).reshape(G, N, N, HC)
    msg = alpha128 * (v[:, :, None, :] + am[..., None] * we.reshape(1, 1, 1, HC))
    out1 = jnp.sum(msg, axis=1).reshape(G * N, HC)

    out_ref[...] = jnp.maximum(out1 + skip, 0.0)


@functools.partial(jax.jit, static_argnames=())
def kernel(node_obs, adj, W1, b1, W2, b2, Wq, bq, Wk, bk, Wv, bv, We,
           Wskip, bskip):
    M, N, D = node_obs.shape
    EH = W2.shape[0]
    HC = We.shape[1]
    G = _G

    w1d = W1[:D]
    w1e = W1[D:D + 1]
    wcat = jnp.concatenate([Wq, Wk, Wv, Wskip], axis=1)
    bcat = jnp.concatenate([bq, bk, bv, bskip]).reshape(1, 4 * HC)
    hs = jnp.asarray(np.kron(np.eye(_H), np.ones((_C, 1))), dtype=jnp.float32)
    hst = jnp.asarray(np.kron(np.eye(_H), np.ones((1, _C))), dtype=jnp.float32)

    full = lambda shape: pl.BlockSpec(shape, lambda i: (0,) * len(shape))
    out = pl.pallas_call(
        _gnn_kernel,
        grid=(M // G,),
        in_specs=[
            pl.BlockSpec((G, N, D), lambda i: (i, 0, 0)),
            pl.BlockSpec((G, N, N), lambda i: (i, 0, 0)),
            full((D, EH)),
            full((1, EH)),
            full((1, EH)),
            full((EH, EH)),
            full((1, EH)),
            full((EH, 4 * HC)),
            full((1, 4 * HC)),
            full((1, HC)),
            full((HC, _H)),
            full((_H, HC)),
        ],
        out_specs=pl.BlockSpec((G * N, HC), lambda i: (i, 0)),
        out_shape=jax.ShapeDtypeStruct((M * N, HC), jnp.float32),
    )(node_obs, adj, w1d, w1e, b1.reshape(1, EH), W2, b2.reshape(1, EH),
      wcat, bcat, We, hs, hst)
    return out


# speedup vs baseline: 535.5144x; 1.5346x over previous
import functools

import numpy as np
import jax
import jax.numpy as jnp
from jax.experimental import pallas as pl

_RADIUS = 0.5
_H = 4
_C = 32
_G = 8


def _sel_pack(n, h, c):
    rows = np.arange(n * h * c)
    d, hh = rows // (h * c), (rows % (h * c)) // c
    m = np.zeros((n * h * c, h * n), np.float32)
    m[rows, hh * n + d] = 1.0
    return m


def _sel_spread(n, h, c):
    m = np.zeros((h * n, n * h * c), np.float32)
    for hh in range(h):
        for d in range(n):
            m[hh * n + d, d * h * c + hh * c:d * h * c + (hh + 1) * c] = 1.0
    return m


def _gnn_kernel(x_ref, a_ref, ap_ref, w1big_ref, w1erep_ref, b1rep_ref,
                w2i_ref, b2rep_ref, wcat_ref, bcat_ref, we_ref, hs_ref,
                hst_ref, hsbig_ref, spread_ref, out_ref):
    G, N, D = x_ref.shape
    EH = 32
    HC = we_ref.shape[1]
    NE = N * EH
    f32 = jnp.float32
    hi = jax.lax.Precision.DEFAULT

    a2 = a_ref[...].reshape(G * N, N)
    maskf2 = ((a2 > 0.0) & (a2 < _RADIUS)).astype(f32)
    am2 = a2 * maskf2

    x2 = x_ref[...].reshape(G * N, D)
    t1w = jnp.dot(x2, w1big_ref[...], precision=hi) + b1rep_ref[...]
    amt = jnp.tile(am2, (1, EH))
    h1w = jnp.maximum(t1w + amt * w1erep_ref[...], 0.0)
    h2w = jnp.maximum(jnp.dot(h1w, w2i_ref[...], precision=hi)
                      + b2rep_ref[...], 0.0)
    h2w = h2w * jnp.tile(maskf2, (1, EH))
    x_embw = jnp.sum(h2w.reshape(G, N, NE), axis=1)
    x_emb = jnp.swapaxes(x_embw.reshape(G, EH, N), 1, 2).reshape(G * N, EH)

    proj = jnp.dot(x_emb, wcat_ref[...], precision=hi) + bcat_ref[...]
    q = proj[:, 0 * HC:1 * HC]
    k = proj[:, 1 * HC:2 * HC].reshape(G, N, HC)
    v = proj[:, 2 * HC:3 * HC].reshape(G, N, HC)
    skip = proj[:, 3 * HC:4 * HC]

    we = we_ref[...]

    ap = ap_ref[...]
    maskfp = ((ap > 0.0) & (ap < _RADIUS)).astype(f32)
    amp = ap * maskfp

    qwe = jnp.dot(q * we, hs_ref[...], precision=hi)
    qwe_p = jnp.swapaxes(qwe.reshape(G, N, _H), 1, 2).reshape(G, _H * N)
    q_wide = q.reshape(G, N * HC)
    prod_w = jnp.tile(k, (1, 1, N)) * q_wide[:, None, :]
    logits_p = jnp.dot(prod_w.reshape(G * N, N * HC), hsbig_ref[...],
                       precision=hi).reshape(G, N, HC)
    scale = 1.0 / np.sqrt(float(_C))
    logits = (logits_p + amp * qwe_p[:, None, :]) * scale

    lm = logits + (maskfp - 1.0) * 1e30
    amax = jnp.max(lm, axis=1, keepdims=True)
    amax = jnp.where(amax > -1e29, amax, 0.0)
    ex = jnp.exp(logits - amax) * maskfp
    den = jnp.sum(ex, axis=1, keepdims=True)
    den = jnp.where(den > 0.0, den, 1.0)
    alpha_p = ex / den

    bw = jnp.sum(alpha_p * amp, axis=1)
    bw4 = jnp.swapaxes(bw.reshape(G, _H, N), 1, 2).reshape(G * N, _H)
    bw128 = jnp.dot(bw4, hst_ref[...], precision=hi)

    alpha_w = jnp.dot(alpha_p.reshape(G * N, HC), spread_ref[...],
                      precision=hi).reshape(G, N, N * HC)
    msg_w = alpha_w * jnp.tile(v, (1, 1, N))
    out_v = jnp.sum(msg_w, axis=1)
    out_v = out_v.reshape(G, N, HC).reshape(G * N, HC)

    out_ref[...] = jnp.maximum(out_v + bw128 * we + skip, 0.0)


@functools.partial(jax.jit, static_argnames=())
def kernel(node_obs, adj, W1, b1, W2, b2, Wq, bq, Wk, bk, Wv, bv, We,
           Wskip, bskip):
    M, N, D = node_obs.shape
    EH = W2.shape[0]
    HC = We.shape[1]
    G = _G

    w1big = jnp.repeat(W1[:D], N, axis=1)
    w1erep = jnp.repeat(W1[D:D + 1], N, axis=1)
    b1rep = jnp.repeat(b1[None, :], N, axis=1)
    w2i = jnp.kron(W2, jnp.eye(N, dtype=jnp.float32))
    b2rep = jnp.repeat(b2[None, :], N, axis=1)
    wcat = jnp.concatenate([Wq, Wk, Wv, Wskip], axis=1)
    bcat = jnp.concatenate([bq, bk, bv, bskip]).reshape(1, 4 * HC)
    hs = jnp.asarray(np.kron(np.eye(_H), np.ones((_C, 1))), jnp.float32)
    hst = jnp.asarray(np.kron(np.eye(_H), np.ones((1, _C))), jnp.float32)
    hsbig = jnp.asarray(_sel_pack(N, _H, _C))
    spread = jnp.asarray(_sel_spread(N, _H, _C))
    adj_ph = jnp.tile(adj, (1, 1, _H))

    full = lambda shape: pl.BlockSpec(shape, lambda i: (0,) * len(shape))
    out = pl.pallas_call(
        _gnn_kernel,
        grid=(M // G,),
        in_specs=[
            pl.BlockSpec((G, N, D), lambda i: (i, 0, 0)),
            pl.BlockSpec((G, N, N), lambda i: (i, 0, 0)),
            pl.BlockSpec((G, N, N * _H), lambda i: (i, 0, 0)),
            full((D, EH * N)),
            full((1, EH * N)),
            full((1, EH * N)),
            full((EH * N, EH * N)),
            full((1, EH * N)),
            full((EH, 4 * HC)),
            full((1, 4 * HC)),
            full((1, HC)),
            full((HC, _H)),
            full((_H, HC)),
            full((N * HC, HC)),
            full((HC, N * HC)),
        ],
        out_specs=pl.BlockSpec((G * N, HC), lambda i: (i, 0)),
        out_shape=jax.ShapeDtypeStruct((M * N, HC), jnp.float32),
    )(node_obs, adj, adj_ph, w1big, w1erep, b1rep, w2i, b2rep, wcat, bcat,
      We, hs, hst, hsbig, spread)
    return out


# speedup vs baseline: 606.5529x; 1.1327x over previous
import functools

import numpy as np
import jax
import jax.numpy as jnp
from jax.experimental import pallas as pl

_RADIUS = 0.5
_H = 4
_C = 32
_G = 16


def _sel_pack(n, h, c):
    rows = np.arange(n * h * c)
    d, hh = rows // (h * c), (rows % (h * c)) // c
    m = np.zeros((n * h * c, h * n), np.float32)
    m[rows, hh * n + d] = 1.0
    return m


def _sel_spread(n, h, c):
    m = np.zeros((h * n, n * h * c), np.float32)
    for hh in range(h):
        for d in range(n):
            m[hh * n + d, d * h * c + hh * c:d * h * c + (hh + 1) * c] = 1.0
    return m


def _gnn_kernel(x_ref, a_ref, ap_ref, w1big_ref, w1erep_ref, b1rep_ref,
                w2i_ref, b2rep_ref, wcat_ref, bcat_ref, we_ref, hs_ref,
                hst_ref, hsbig_ref, spread_ref, out_ref):
    G, N, D = x_ref.shape
    EH = 32
    HC = we_ref.shape[1]
    NE = N * EH
    f32 = jnp.float32
    hi = jax.lax.Precision.DEFAULT

    a2 = a_ref[...].reshape(G * N, N)
    maskf2 = ((a2 > 0.0) & (a2 < _RADIUS)).astype(f32)
    am2 = a2 * maskf2

    x2 = x_ref[...].reshape(G * N, D)
    t1w = jnp.dot(x2, w1big_ref[...], precision=hi) + b1rep_ref[...]
    amt = jnp.tile(am2, (1, EH))
    h1w = jnp.maximum(t1w + amt * w1erep_ref[...], 0.0)
    h2w = jnp.maximum(jnp.dot(h1w, w2i_ref[...], precision=hi)
                      + b2rep_ref[...], 0.0)
    h2w = h2w * jnp.tile(maskf2, (1, EH))
    x_embw = jnp.sum(h2w.reshape(G, N, NE), axis=1)
    x_emb = jnp.swapaxes(x_embw.reshape(G, EH, N), 1, 2).reshape(G * N, EH)

    proj = jnp.dot(x_emb, wcat_ref[...], precision=hi) + bcat_ref[...]
    q = proj[:, 0 * HC:1 * HC]
    k = proj[:, 1 * HC:2 * HC].reshape(G, N, HC)
    v = proj[:, 2 * HC:3 * HC].reshape(G, N, HC)
    skip = proj[:, 3 * HC:4 * HC]

    we = we_ref[...]

    ap = ap_ref[...]
    maskfp = ((ap > 0.0) & (ap < _RADIUS)).astype(f32)
    amp = ap * maskfp

    qwe = jnp.dot(q * we, hs_ref[...], precision=hi)
    qwe_p = jnp.swapaxes(qwe.reshape(G, N, _H), 1, 2).reshape(G, _H * N)
    q_wide = q.reshape(G, N * HC)
    prod_w = jnp.tile(k, (1, 1, N)) * q_wide[:, None, :]
    logits_p = jnp.dot(prod_w.reshape(G * N, N * HC), hsbig_ref[...],
                       precision=hi).reshape(G, N, HC)
    scale = 1.0 / np.sqrt(float(_C))
    logits = (logits_p + amp * qwe_p[:, None, :]) * scale

    lm = logits + (maskfp - 1.0) * 1e30
    amax = jnp.max(lm, axis=1, keepdims=True)
    amax = jnp.where(amax > -1e29, amax, 0.0)
    ex = jnp.exp(logits - amax) * maskfp
    den = jnp.sum(ex, axis=1, keepdims=True)
    den = jnp.where(den > 0.0, den, 1.0)
    alpha_p = ex / den

    bw = jnp.sum(alpha_p * amp, axis=1)
    bw4 = jnp.swapaxes(bw.reshape(G, _H, N), 1, 2).reshape(G * N, _H)
    bw128 = jnp.dot(bw4, hst_ref[...], precision=hi)

    alpha_w = jnp.dot(alpha_p.reshape(G * N, HC), spread_ref[...],
                      precision=hi).reshape(G, N, N * HC)
    msg_w = alpha_w * jnp.tile(v, (1, 1, N))
    out_v = jnp.sum(msg_w, axis=1)
    out_v = out_v.reshape(G, N, HC).reshape(G * N, HC)

    out_ref[...] = jnp.maximum(out_v + bw128 * we + skip, 0.0)


@functools.partial(jax.jit, static_argnames=())
def kernel(node_obs, adj, W1, b1, W2, b2, Wq, bq, Wk, bk, Wv, bv, We,
           Wskip, bskip):
    M, N, D = node_obs.shape
    EH = W2.shape[0]
    HC = We.shape[1]
    G = _G

    w1big = jnp.repeat(W1[:D], N, axis=1)
    w1erep = jnp.repeat(W1[D:D + 1], N, axis=1)
    b1rep = jnp.repeat(b1[None, :], N, axis=1)
    w2i = jnp.kron(W2, jnp.eye(N, dtype=jnp.float32))
    b2rep = jnp.repeat(b2[None, :], N, axis=1)
    wcat = jnp.concatenate([Wq, Wk, Wv, Wskip], axis=1)
    bcat = jnp.concatenate([bq, bk, bv, bskip]).reshape(1, 4 * HC)
    hs = jnp.asarray(np.kron(np.eye(_H), np.ones((_C, 1))), jnp.float32)
    hst = jnp.asarray(np.kron(np.eye(_H), np.ones((1, _C))), jnp.float32)
    hsbig = jnp.asarray(_sel_pack(N, _H, _C))
    spread = jnp.asarray(_sel_spread(N, _H, _C))
    adj_ph = jnp.tile(adj, (1, 1, _H))

    full = lambda shape: pl.BlockSpec(shape, lambda i: (0,) * len(shape))
    out = pl.pallas_call(
        _gnn_kernel,
        grid=(M // G,),
        in_specs=[
            pl.BlockSpec((G, N, D), lambda i: (i, 0, 0)),
            pl.BlockSpec((G, N, N), lambda i: (i, 0, 0)),
            pl.BlockSpec((G, N, N * _H), lambda i: (i, 0, 0)),
            full((D, EH * N)),
            full((1, EH * N)),
            full((1, EH * N)),
            full((EH * N, EH * N)),
            full((1, EH * N)),
            full((EH, 4 * HC)),
            full((1, 4 * HC)),
            full((1, HC)),
            full((HC, _H)),
            full((_H, HC)),
            full((N * HC, HC)),
            full((HC, N * HC)),
        ],
        out_specs=pl.BlockSpec((G * N, HC), lambda i: (i, 0)),
        out_shape=jax.ShapeDtypeStruct((M * N, HC), jnp.float32),
    )(node_obs, adj, adj_ph, w1big, w1erep, b1rep, w2i, b2rep, wcat, bcat,
      We, hs, hst, hsbig, spread)
    return out


# speedup vs baseline: 656.1566x; 1.0818x over previous
import functools

import numpy as np
import jax
import jax.numpy as jnp
from jax.experimental import pallas as pl

_RADIUS = 0.5
_H = 4
_C = 32
_G = 16


def _sel_pack(n, h, c):
    rows = np.arange(n * h * c)
    d, hh = rows // (h * c), (rows % (h * c)) // c
    m = np.zeros((n * h * c, h * n), np.float32)
    m[rows, hh * n + d] = 1.0
    return m


def _sel_spread(n, h, c):
    m = np.zeros((h * n, n * h * c), np.float32)
    for hh in range(h):
        for d in range(n):
            m[hh * n + d, d * h * c + hh * c:d * h * c + (hh + 1) * c] = 1.0
    return m


def _gnn_kernel(x_ref, a_ref, w1big_ref, w1erep_ref, b1rep_ref,
                w2i_ref, b2rep_ref, wcat_ref, bcat_ref, we_ref, hs_ref,
                hst_ref, hsbig_ref, spread_ref, out_ref):
    G, N, D = x_ref.shape
    EH = 32
    HC = we_ref.shape[1]
    NE = N * EH
    f32 = jnp.float32
    hi = jax.lax.Precision.DEFAULT

    a2 = a_ref[...].reshape(G * N, N)
    maskf2 = ((a2 > 0.0) & (a2 < _RADIUS)).astype(f32)
    am2 = a2 * maskf2

    x2 = x_ref[...].reshape(G * N, D)
    t1w = jnp.dot(x2, w1big_ref[...], precision=hi) + b1rep_ref[...]
    amt = jnp.tile(am2, (1, EH))
    h1w = jnp.maximum(t1w + amt * w1erep_ref[...], 0.0)
    h2w = jnp.maximum(jnp.dot(h1w, w2i_ref[...], precision=hi)
                      + b2rep_ref[...], 0.0)
    h2w = h2w * jnp.tile(maskf2, (1, EH))
    x_embw = jnp.sum(h2w.reshape(G, N, NE), axis=1)
    x_emb = jnp.swapaxes(x_embw.reshape(G, EH, N), 1, 2).reshape(G * N, EH)

    proj = jnp.dot(x_emb, wcat_ref[...], precision=hi) + bcat_ref[...]
    q = proj[:, 0 * HC:1 * HC]
    k = proj[:, 1 * HC:2 * HC].reshape(G, N, HC)
    v = proj[:, 2 * HC:3 * HC].reshape(G, N, HC)
    skip = proj[:, 3 * HC:4 * HC]

    we = we_ref[...]

    maskfp = jnp.tile(maskf2, (1, _H)).reshape(G, N, HC)
    amp = jnp.tile(am2, (1, _H)).reshape(G, N, HC)

    qwe = jnp.dot(q * we, hs_ref[...], precision=hi)
    qwe_p = jnp.swapaxes(qwe.reshape(G, N, _H), 1, 2).reshape(G, _H * N)
    q_wide = q.reshape(G, N * HC)
    prod_w = jnp.tile(k, (1, 1, N)) * q_wide[:, None, :]
    logits_p = jnp.dot(prod_w.reshape(G * N, N * HC), hsbig_ref[...],
                       precision=hi).reshape(G, N, HC)
    scale = 1.0 / np.sqrt(float(_C))
    logits = (logits_p + amp * qwe_p[:, None, :]) * scale

    lm = logits + (maskfp - 1.0) * 1e30
    amax = jnp.max(lm, axis=1, keepdims=True)
    amax = jnp.where(amax > -1e29, amax, 0.0)
    ex = jnp.exp(logits - amax) * maskfp
    den = jnp.sum(ex, axis=1, keepdims=True)
    den = jnp.where(den > 0.0, den, 1.0)
    alpha_p = ex / den

    bw = jnp.sum(alpha_p * amp, axis=1)
    bw4 = jnp.swapaxes(bw.reshape(G, _H, N), 1, 2).reshape(G * N, _H)
    bw128 = jnp.dot(bw4, hst_ref[...], precision=hi)

    alpha_w = jnp.dot(alpha_p.reshape(G * N, HC), spread_ref[...],
                      precision=hi).reshape(G, N, N * HC)
    msg_w = alpha_w * jnp.tile(v, (1, 1, N))
    out_v = jnp.sum(msg_w, axis=1)
    out_v = out_v.reshape(G, N, HC).reshape(G * N, HC)

    out_ref[...] = jnp.maximum(out_v + bw128 * we + skip, 0.0)


@functools.partial(jax.jit, static_argnames=())
def kernel(node_obs, adj, W1, b1, W2, b2, Wq, bq, Wk, bk, Wv, bv, We,
           Wskip, bskip):
    M, N, D = node_obs.shape
    EH = W2.shape[0]
    HC = We.shape[1]
    G = _G

    w1big = jnp.repeat(W1[:D], N, axis=1)
    w1erep = jnp.repeat(W1[D:D + 1], N, axis=1)
    b1rep = jnp.repeat(b1[None, :], N, axis=1)
    w2i = jnp.kron(W2, jnp.eye(N, dtype=jnp.float32))
    b2rep = jnp.repeat(b2[None, :], N, axis=1)
    wcat = jnp.concatenate([Wq, Wk, Wv, Wskip], axis=1)
    bcat = jnp.concatenate([bq, bk, bv, bskip]).reshape(1, 4 * HC)
    hs = jnp.asarray(np.kron(np.eye(_H), np.ones((_C, 1))), jnp.float32)
    hst = jnp.asarray(np.kron(np.eye(_H), np.ones((1, _C))), jnp.float32)
    hsbig = jnp.asarray(_sel_pack(N, _H, _C))
    spread = jnp.asarray(_sel_spread(N, _H, _C))

    full = lambda shape: pl.BlockSpec(shape, lambda i: (0,) * len(shape))
    out = pl.pallas_call(
        _gnn_kernel,
        grid=(M // G,),
        in_specs=[
            pl.BlockSpec((G, N, D), lambda i: (i, 0, 0)),
            pl.BlockSpec((G, N, N), lambda i: (i, 0, 0)),
            full((D, EH * N)),
            full((1, EH * N)),
            full((1, EH * N)),
            full((EH * N, EH * N)),
            full((1, EH * N)),
            full((EH, 4 * HC)),
            full((1, 4 * HC)),
            full((1, HC)),
            full((HC, _H)),
            full((_H, HC)),
            full((N * HC, HC)),
            full((HC, N * HC)),
        ],
        out_specs=pl.BlockSpec((G * N, HC), lambda i: (i, 0)),
        out_shape=jax.ShapeDtypeStruct((M * N, HC), jnp.float32),
    )(node_obs, adj, w1big, w1erep, b1rep, w2i, b2rep, wcat, bcat,
      We, hs, hst, hsbig, spread)
    return out


# speedup vs baseline: 738.5176x; 1.1255x over previous
import functools

import numpy as np
import jax
import jax.numpy as jnp
from jax.experimental import pallas as pl
from jax.experimental.pallas import tpu as pltpu

_RADIUS = 0.5
_H = 4
_C = 32
_G = 16


def _sel_pack(n, h, c):
    rows = np.arange(n * h * c)
    d, hh = rows // (h * c), (rows % (h * c)) // c
    m = np.zeros((n * h * c, h * n), np.float32)
    m[rows, hh * n + d] = 1.0
    return m


def _sel_spread(n, h, c):
    m = np.zeros((h * n, n * h * c), np.float32)
    for hh in range(h):
        for d in range(n):
            m[hh * n + d, d * h * c + hh * c:d * h * c + (hh + 1) * c] = 1.0
    return m


def _gnn_kernel(x_ref, a_ref, w1_ref, b1_ref, w2_ref, b2_ref, wq_ref,
                bq_ref, wk_ref, bk_ref, wv_ref, bv_ref, we_ref, wskip_ref,
                bskip_ref, hs_ref, hst_ref, hsbig_ref, spread_ref, out_ref,
                w1big_s, misc_s, w2i_s, wcat_s, bcat_s):
    G, N, D = x_ref.shape
    EH = 32
    HC = we_ref.shape[1]
    NE = N * EH
    f32 = jnp.float32
    hi = jax.lax.Precision.DEFAULT

    @pl.when(pl.program_id(0) == 0)
    def _setup():
        w1big_s[...] = jnp.repeat(w1_ref[0:D, :], N, axis=1)
        misc_s[0:1, :] = jnp.repeat(w1_ref[D:D + 1, :], N, axis=1)
        misc_s[1:2, :] = jnp.repeat(b1_ref[...], N, axis=1)
        misc_s[2:3, :] = jnp.repeat(b2_ref[...], N, axis=1)
        w2rep = jnp.repeat(jnp.repeat(w2_ref[...], N, axis=0), N, axis=1)
        rr = jax.lax.broadcasted_iota(jnp.int32, (NE, NE), 0) % N
        cc = jax.lax.broadcasted_iota(jnp.int32, (NE, NE), 1) % N
        w2i_s[...] = w2rep * (rr == cc).astype(f32)
        wcat_s[...] = jnp.concatenate(
            [wq_ref[...], wk_ref[...], wv_ref[...], wskip_ref[...]], axis=1)
        bcat_s[0:1, :] = jnp.concatenate(
            [bq_ref[...], bk_ref[...], bv_ref[...], bskip_ref[...]], axis=1)

    a2 = a_ref[...].reshape(G * N, N)
    maskf2 = ((a2 > 0.0) & (a2 < _RADIUS)).astype(f32)
    am2 = a2 * maskf2

    x2 = x_ref[...].reshape(G * N, D)
    t1w = jnp.dot(x2, w1big_s[...], precision=hi) + misc_s[1:2, :]
    amt = jnp.tile(am2, (1, EH))
    h1w = jnp.maximum(t1w + amt * misc_s[0:1, :], 0.0)
    h2w = jnp.maximum(jnp.dot(h1w, w2i_s[...], precision=hi)
                      + misc_s[2:3, :], 0.0)
    h2w = h2w * jnp.tile(maskf2, (1, EH))
    x_embw = jnp.sum(h2w.reshape(G, N, NE), axis=1)
    x_emb = jnp.swapaxes(x_embw.reshape(G, EH, N), 1, 2).reshape(G * N, EH)

    proj = jnp.dot(x_emb, wcat_s[...], precision=hi) + bcat_s[0:1, :]
    q = proj[:, 0 * HC:1 * HC]
    k = proj[:, 1 * HC:2 * HC].reshape(G, N, HC)
    v = proj[:, 2 * HC:3 * HC].reshape(G, N, HC)
    skip = proj[:, 3 * HC:4 * HC]

    we = we_ref[...]

    maskfp = jnp.tile(maskf2, (1, _H)).reshape(G, N, HC)
    amp = jnp.tile(am2, (1, _H)).reshape(G, N, HC)

    qwe = jnp.dot(q * we, hs_ref[...], precision=hi)
    qwe_p = jnp.swapaxes(qwe.reshape(G, N, _H), 1, 2).reshape(G, _H * N)
    q_wide = q.reshape(G, N * HC)
    prod_w = jnp.tile(k, (1, 1, N)) * q_wide[:, None, :]
    logits_p = jnp.dot(prod_w.reshape(G * N, N * HC), hsbig_ref[...],
                       precision=hi).reshape(G, N, HC)
    scale = 1.0 / np.sqrt(float(_C))
    logits = (logits_p + amp * qwe_p[:, None, :]) * scale

    lm = logits + (maskfp - 1.0) * 1e30
    amax = jnp.max(lm, axis=1, keepdims=True)
    amax = jnp.where(amax > -1e29, amax, 0.0)
    ex = jnp.exp(logits - amax) * maskfp
    den = jnp.sum(ex, axis=1, keepdims=True)
    den = jnp.where(den > 0.0, den, 1.0)
    alpha_p = ex / den

    bw = jnp.sum(alpha_p * amp, axis=1)
    bw4 = jnp.swapaxes(bw.reshape(G, _H, N), 1, 2).reshape(G * N, _H)
    bw128 = jnp.dot(bw4, hst_ref[...], precision=hi)

    alpha_w = jnp.dot(alpha_p.reshape(G * N, HC), spread_ref[...],
                      precision=hi).reshape(G, N, N * HC)
    msg_w = alpha_w * jnp.tile(v, (1, 1, N))
    out_v = jnp.sum(msg_w, axis=1)
    out_v = out_v.reshape(G, N, HC).reshape(G * N, HC)

    out_ref[...] = jnp.maximum(out_v + bw128 * we + skip, 0.0)


@functools.partial(jax.jit, static_argnames=())
def kernel(node_obs, adj, W1, b1, W2, b2, Wq, bq, Wk, bk, Wv, bv, We,
           Wskip, bskip):
    M, N, D = node_obs.shape
    EH = W2.shape[0]
    HC = We.shape[1]
    G = _G
    f32 = jnp.float32

    hs = jnp.asarray(np.kron(np.eye(_H), np.ones((_C, 1))), f32)
    hst = jnp.asarray(np.kron(np.eye(_H), np.ones((1, _C))), f32)
    hsbig = jnp.asarray(_sel_pack(N, _H, _C))
    spread = jnp.asarray(_sel_spread(N, _H, _C))

    full = lambda shape: pl.BlockSpec(shape, lambda i: (0,) * len(shape))
    out = pl.pallas_call(
        _gnn_kernel,
        grid=(M // G,),
        in_specs=[
            pl.BlockSpec((G, N, D), lambda i: (i, 0, 0)),
            pl.BlockSpec((G, N, N), lambda i: (i, 0, 0)),
            full((D + 1, EH)),
            full((1, EH)),
            full((EH, EH)),
            full((1, EH)),
            full((EH, HC)),
            full((1, HC)),
            full((EH, HC)),
            full((1, HC)),
            full((EH, HC)),
            full((1, HC)),
            full((1, HC)),
            full((EH, HC)),
            full((1, HC)),
            full((HC, _H)),
            full((_H, HC)),
            full((N * HC, HC)),
            full((HC, N * HC)),
        ],
        out_specs=pl.BlockSpec((G * N, HC), lambda i: (i, 0)),
        out_shape=jax.ShapeDtypeStruct((M * N, HC), jnp.float32),
        scratch_shapes=[
            pltpu.VMEM((D, EH * N), f32),
            pltpu.VMEM((8, EH * N), f32),
            pltpu.VMEM((EH * N, EH * N), f32),
            pltpu.VMEM((EH, 4 * HC), f32),
            pltpu.VMEM((8, 4 * HC), f32),
        ],
    )(node_obs, adj, W1, b1.reshape(1, EH), W2, b2.reshape(1, EH),
      Wq, bq.reshape(1, HC), Wk, bk.reshape(1, HC), Wv, bv.reshape(1, HC),
      We, Wskip, bskip.reshape(1, HC), hs, hst, hsbig, spread)
    return out


# speedup vs baseline: 853.8863x; 1.1562x over previous
import functools

import numpy as np
import jax
import jax.numpy as jnp
from jax.experimental import pallas as pl
from jax.experimental.pallas import tpu as pltpu

_RADIUS = 0.5
_H = 4
_C = 32
_G = 16


def _sel_pack(n, h, c):
    rows = np.arange(n * h * c)
    d, hh = rows // (h * c), (rows % (h * c)) // c
    m = np.zeros((n * h * c, h * n), np.float32)
    m[rows, hh * n + d] = 1.0
    return m


def _sel_spread(n, h, c):
    m = np.zeros((h * n, n * h * c), np.float32)
    for hh in range(h):
        for d in range(n):
            m[hh * n + d, d * h * c + hh * c:d * h * c + (hh + 1) * c] = 1.0
    return m


def _gnn_kernel(x_ref, a_ref, w1_ref, b1_ref, w2_ref, b2_ref, wq_ref,
                bq_ref, wk_ref, bk_ref, wv_ref, bv_ref, we_ref, wskip_ref,
                bskip_ref, hs_ref, hst_ref, out_ref,
                w1big_s, misc_s, w2i_s, wcat_s, bcat_s):
    G, N, D = x_ref.shape
    EH = 32
    HC = we_ref.shape[1]
    NE = N * EH
    f32 = jnp.float32
    hi = jax.lax.Precision.DEFAULT

    @pl.when(pl.program_id(0) == 0)
    def _setup():
        w1big_s[...] = jnp.repeat(w1_ref[0:D, :], N, axis=1)
        misc_s[0:1, :] = jnp.repeat(w1_ref[D:D + 1, :], N, axis=1)
        misc_s[1:2, :] = jnp.repeat(b1_ref[...], N, axis=1)
        misc_s[2:3, :] = jnp.repeat(b2_ref[...], N, axis=1)
        w2rep = jnp.repeat(jnp.repeat(w2_ref[...], N, axis=0), N, axis=1)
        rr = jax.lax.broadcasted_iota(jnp.int32, (NE, NE), 0) % N
        cc = jax.lax.broadcasted_iota(jnp.int32, (NE, NE), 1) % N
        w2i_s[...] = w2rep * (rr == cc).astype(f32)
        wcat_s[...] = jnp.concatenate(
            [wq_ref[...], wk_ref[...], wv_ref[...], wskip_ref[...]], axis=1)
        bcat_s[0:1, :] = jnp.concatenate(
            [bq_ref[...], bk_ref[...], bv_ref[...], bskip_ref[...]], axis=1)

    a2 = a_ref[...].reshape(G * N, N)
    maskf2 = ((a2 > 0.0) & (a2 < _RADIUS)).astype(f32)
    am2 = a2 * maskf2

    x2 = x_ref[...].reshape(G * N, D)
    t1w = jnp.dot(x2, w1big_s[...], precision=hi) + misc_s[1:2, :]
    amt = jnp.tile(am2, (1, EH))
    h1w = jnp.maximum(t1w + amt * misc_s[0:1, :], 0.0)
    h2w = jnp.maximum(jnp.dot(h1w, w2i_s[...], precision=hi)
                      + misc_s[2:3, :], 0.0)
    h2w = h2w * jnp.tile(maskf2, (1, EH))
    x_embw = jnp.sum(h2w.reshape(G, N, NE), axis=1)
    x_emb = jnp.swapaxes(x_embw.reshape(G, EH, N), 1, 2).reshape(G * N, EH)

    proj = jnp.dot(x_emb, wcat_s[...], precision=hi) + bcat_s[0:1, :]
    q = proj[:, 0 * HC:1 * HC]
    k = proj[:, 1 * HC:2 * HC].reshape(G, N, HC)
    v = proj[:, 2 * HC:3 * HC].reshape(G, N, HC)
    skip = proj[:, 3 * HC:4 * HC]

    we = we_ref[...]

    maskfp = jnp.tile(maskf2, (1, _H)).reshape(G, N, HC)
    amp = jnp.tile(am2, (1, _H)).reshape(G, N, HC)

    qwe = jnp.dot(q * we, hs_ref[...], precision=hi)
    qwe_p = jnp.swapaxes(qwe.reshape(G, N, _H), 1, 2).reshape(G, _H * N)
    qT = jnp.swapaxes(q.reshape(G, N, HC), 1, 2)
    logits_p = jnp.stack([
        jnp.concatenate([
            jnp.dot(k[g, :, h * _C:(h + 1) * _C],
                    qT[g, h * _C:(h + 1) * _C, :], precision=hi)
            for h in range(_H)], axis=1)
        for g in range(G)], axis=0)
    scale = 1.0 / np.sqrt(float(_C))
    logits = (logits_p + amp * qwe_p[:, None, :]) * scale

    lm = logits + (maskfp - 1.0) * 1e30
    amax = jnp.max(lm, axis=1, keepdims=True)
    amax = jnp.where(amax > -1e29, amax, 0.0)
    ex = jnp.exp(logits - amax) * maskfp
    den = jnp.sum(ex, axis=1, keepdims=True)
    den = jnp.where(den > 0.0, den, 1.0)
    alpha_p = ex / den

    bw = jnp.sum(alpha_p * amp, axis=1)
    bw4 = jnp.swapaxes(bw.reshape(G, _H, N), 1, 2).reshape(G * N, _H)
    bw128 = jnp.dot(bw4, hst_ref[...], precision=hi)

    alphaT = jnp.swapaxes(alpha_p, 1, 2)
    out_v = jnp.stack([
        jnp.concatenate([
            jnp.dot(alphaT[g, h * N:(h + 1) * N, :],
                    v[g, :, h * _C:(h + 1) * _C], precision=hi)
            for h in range(_H)], axis=1)
        for g in range(G)], axis=0)
    out_v = out_v.reshape(G * N, HC)

    out_ref[...] = jnp.maximum(out_v + bw128 * we + skip, 0.0)


@functools.partial(jax.jit, static_argnames=())
def kernel(node_obs, adj, W1, b1, W2, b2, Wq, bq, Wk, bk, Wv, bv, We,
           Wskip, bskip):
    M, N, D = node_obs.shape
    EH = W2.shape[0]
    HC = We.shape[1]
    G = _G
    f32 = jnp.float32

    hs = jnp.asarray(np.kron(np.eye(_H), np.ones((_C, 1))), f32)
    hst = jnp.asarray(np.kron(np.eye(_H), np.ones((1, _C))), f32)

    full = lambda shape: pl.BlockSpec(shape, lambda i: (0,) * len(shape))
    out = pl.pallas_call(
        _gnn_kernel,
        grid=(M // G,),
        in_specs=[
            pl.BlockSpec((G, N, D), lambda i: (i, 0, 0)),
            pl.BlockSpec((G, N, N), lambda i: (i, 0, 0)),
            full((D + 1, EH)),
            full((1, EH)),
            full((EH, EH)),
            full((1, EH)),
            full((EH, HC)),
            full((1, HC)),
            full((EH, HC)),
            full((1, HC)),
            full((EH, HC)),
            full((1, HC)),
            full((1, HC)),
            full((EH, HC)),
            full((1, HC)),
            full((HC, _H)),
            full((_H, HC)),
        ],
        out_specs=pl.BlockSpec((G * N, HC), lambda i: (i, 0)),
        out_shape=jax.ShapeDtypeStruct((M * N, HC), jnp.float32),
        scratch_shapes=[
            pltpu.VMEM((D, EH * N), f32),
            pltpu.VMEM((8, EH * N), f32),
            pltpu.VMEM((EH * N, EH * N), f32),
            pltpu.VMEM((EH, 4 * HC), f32),
            pltpu.VMEM((8, 4 * HC), f32),
        ],
    )(node_obs, adj, W1, b1.reshape(1, EH), W2, b2.reshape(1, EH),
      Wq, bq.reshape(1, HC), Wk, bk.reshape(1, HC), Wv, bv.reshape(1, HC),
      We, Wskip, bskip.reshape(1, HC), hs, hst)
    return out
